# baseline (speedup 1.0000x reference)
import jax
import jax.numpy as jnp
from jax.experimental import pallas as pl
from jax.experimental.pallas import tpu as pltpu

B, L, V, E, H = 16384, 20, 512, 64, 64
BLK = 2048
SUB = 1024
NB = B // BLK


def _body(lens, ids_e, len_e, ids_d, len_d, emb, wie, whe, bie,
          wid, whd, bid, bhe, bhd, out, ge_ref, gd_ref, h_ref):
    i = pl.program_id(0)
    me = lens[i, 0]
    md = lens[i, 1]

    @pl.when(i == 0)
    def _build_tables():
        embv = emb[:].astype(jnp.bfloat16)
        ge_ref[:] = (jax.lax.dot_general(
            embv, wie[:].astype(jnp.bfloat16), (((1,), (1,)), ((), ())),
            preferred_element_type=jnp.float32) + bie[:]).astype(jnp.bfloat16)
        gd_ref[:] = (jax.lax.dot_general(
            embv, wid[:].astype(jnp.bfloat16), (((1,), (1,)), ((), ())),
            preferred_element_type=jnp.float32) + bid[:]).astype(jnp.bfloat16)

    iota16 = jax.lax.broadcasted_iota(jnp.int32, (SUB, V), 1).astype(jnp.int16)
    one_bf = jnp.ones((), jnp.bfloat16)
    zero_bf = jnp.zeros((), jnp.bfloat16)

    whe_ = whe[:].astype(jnp.bfloat16)
    whd_ = whd[:].astype(jnp.bfloat16)
    bheb = jnp.broadcast_to(bhe[:], (SUB, 3 * H))
    bhdb = jnp.broadcast_to(bhd[:], (SUB, 3 * H))

    lebA = jnp.broadcast_to(len_e[0:SUB, :], (SUB, H))
    lebB = jnp.broadcast_to(len_e[SUB:BLK, :], (SUB, H))
    ldbA = jnp.broadcast_to(len_d[0:SUB, :], (SUB, H))
    ldbB = jnp.broadcast_to(len_d[SUB:BLK, :], (SUB, H))

    def cell(ids_ref, g_ref, a0, t, wh, bhb, h):
        idc = ids_ref[a0:a0 + SUB, t:t + 1]
        oh = jnp.where(idc == iota16, one_bf, zero_bf)
        gi = jax.lax.dot_general(oh, g_ref[:], (((1,), (0,)), ((), ())),
                                 preferred_element_type=jnp.float32)
        gh = jax.lax.dot_general(h.astype(jnp.bfloat16), wh,
                                 (((1,), (1,)), ((), ())),
                                 preferred_element_type=jnp.float32) + bhb
        rz = jax.nn.sigmoid(gi[:, :2 * H] + gh[:, :2 * H])
        z = rz[:, H:]
        n = jnp.tanh(gi[:, 2 * H:] + rz[:, :H] * gh[:, 2 * H:])
        return (1.0 - z) * n + z * h

    def enc_step(t):
        @pl.when(t < me)
        def _():
            for a0, leb in ((0, lebA), (SUB, lebB)):
                h = h_ref[a0:a0 + SUB, :]
                h_new = cell(ids_e, ge_ref, a0, t, whe_, bheb, h)
                h_ref[a0:a0 + SUB, :] = jnp.where(leb > t, h_new, h)

    def dec_step(t):
        @pl.when(t < md)
        def _():
            for a0, ldb in ((0, ldbA), (SUB, ldbB)):
                h = h_ref[a0:a0 + SUB, :]
                h_new = cell(ids_d, gd_ref, a0, t, whd_, bhdb, h)
                msk = ldb > t
                out[a0:a0 + SUB, t * H:(t + 1) * H] = jnp.where(
                    msk, h_new, 0.0)
                h_ref[a0:a0 + SUB, :] = jnp.where(msk, h_new, h)

        @pl.when(t >= md)
        def _():
            out[:, t * H:(t + 1) * H] = jnp.zeros((BLK, H), jnp.float32)

    h_ref[:] = jnp.zeros((BLK, H), jnp.float32)
    for t in range(L):
        enc_step(t)
    for t in range(L):
        dec_step(t)


def _stable_desc_perm(lengths):
    key = (L - lengths).astype(jnp.int32)
    oh = (key[:, None] == jnp.arange(L, dtype=jnp.int32)[None, :]).astype(
        jnp.int32)
    csum = jnp.cumsum(oh, axis=0)
    counts = csum[-1]
    starts = jnp.concatenate(
        [jnp.zeros((1,), jnp.int32), jnp.cumsum(counts)[:-1]])
    within = jnp.take_along_axis(csum, key[:, None], axis=1)[:, 0] - 1
    rank = starts[key] + within
    return jnp.zeros((B,), jnp.int32).at[rank].set(
        jnp.arange(B, dtype=jnp.int32))


def kernel(input, output, input_mask, output_mask, input_word_len,
           output_word_len, emb, w_ih_enc, w_hh_enc, b_ih_enc, b_hh_enc,
           w_ih_dec, w_hh_dec, b_ih_dec, b_hh_dec):
    in_len = input_word_len[:, 0]
    perm_in = _stable_desc_perm(in_len)
    ids_e = jnp.take(input, perm_in, axis=0).astype(jnp.int16)
    len_e = jnp.take(in_len, perm_in).astype(jnp.int32)

    out_len = output_word_len[:, 0]
    perm_out = _stable_desc_perm(out_len)
    ids_d = jnp.take(output, perm_out, axis=0).astype(jnp.int16)
    len_d = jnp.take(out_len, perm_out).astype(jnp.int32)

    maxlens = jnp.stack([len_e[::BLK], len_d[::BLK]], axis=1).astype(jnp.int32)

    full = lambda shape: pl.BlockSpec(shape, lambda i, *_: (0,) * len(shape))
    ids_spec = pl.BlockSpec((BLK, L), lambda i, *_: (i, 0))
    len_spec = pl.BlockSpec((BLK, 1), lambda i, *_: (i, 0))

    grid_spec = pltpu.PrefetchScalarGridSpec(
        num_scalar_prefetch=1,
        grid=(NB,),
        in_specs=[
            ids_spec, len_spec, ids_spec, len_spec,
            full((V, E)),
            full((3 * H, E)), full((3 * H, H)), full((1, 3 * H)),
            full((3 * H, E)), full((3 * H, H)), full((1, 3 * H)),
            full((1, 3 * H)), full((1, 3 * H)),
        ],
        out_specs=pl.BlockSpec((BLK, L * H), lambda i, *_: (i, 0)),
        scratch_shapes=[
            pltpu.VMEM((V, 3 * H), jnp.bfloat16),
            pltpu.VMEM((V, 3 * H), jnp.bfloat16),
            pltpu.VMEM((BLK, H), jnp.float32),
        ],
    )

    out_flat = pl.pallas_call(
        _body,
        grid_spec=grid_spec,
        out_shape=jax.ShapeDtypeStruct((B, L * H), jnp.float32),
    )(maxlens, ids_e, len_e[:, None], ids_d, len_d[:, None], emb,
      w_ih_enc, w_hh_enc, b_ih_enc.reshape(1, 3 * H),
      w_ih_dec, w_hh_dec, b_ih_dec.reshape(1, 3 * H),
      b_hh_enc.reshape(1, 3 * H), b_hh_dec.reshape(1, 3 * H))
    return out_flat.reshape(B, L, H)


# speedup vs baseline: 1.4617x; 1.4617x over previous
import jax
import jax.numpy as jnp
from jax.experimental import pallas as pl
from jax.experimental.pallas import tpu as pltpu

B, L, V, E, H = 16384, 20, 512, 64, 64
BLK = 2048
SUB = 1024
NB = B // BLK


def _body(lens, ids_e, len_e, ids_d, len_d, emb, wie, whe, bie,
          wid, whd, bid, bhe, bhd, out, ge_ref, gd_ref, h_ref):
    i = pl.program_id(0)
    me = lens[i, 0]
    md = lens[i, 1]

    @pl.when(i == 0)
    def _build_tables():
        embv = emb[:].astype(jnp.bfloat16)
        ge_ref[:] = (jax.lax.dot_general(
            embv, wie[:].astype(jnp.bfloat16), (((1,), (1,)), ((), ())),
            preferred_element_type=jnp.float32) + bie[:]).astype(jnp.bfloat16)
        gd_ref[:] = (jax.lax.dot_general(
            embv, wid[:].astype(jnp.bfloat16), (((1,), (1,)), ((), ())),
            preferred_element_type=jnp.float32) + bid[:]).astype(jnp.bfloat16)

    iota16 = jax.lax.broadcasted_iota(jnp.int32, (SUB, V), 1).astype(jnp.int16)
    one_bf = jnp.ones((), jnp.bfloat16)
    zero_bf = jnp.zeros((), jnp.bfloat16)

    whe_ = whe[:].astype(jnp.bfloat16)
    whd_ = whd[:].astype(jnp.bfloat16)
    bheb = jnp.broadcast_to(bhe[:], (SUB, 3 * H))
    bhdb = jnp.broadcast_to(bhd[:], (SUB, 3 * H))

    lebA = jnp.broadcast_to(len_e[0:SUB, :], (SUB, H))
    lebB = jnp.broadcast_to(len_e[SUB:BLK, :], (SUB, H))
    ldbA = jnp.broadcast_to(len_d[0:SUB, :], (SUB, H))
    ldbB = jnp.broadcast_to(len_d[SUB:BLK, :], (SUB, H))

    def cell(ids_ref, g_ref, a0, t, wh, bhb, h):
        idc = ids_ref[a0:a0 + SUB, t:t + 1]
        oh = jnp.where(idc == iota16, one_bf, zero_bf)
        gi = jax.lax.dot_general(oh, g_ref[:], (((1,), (0,)), ((), ())),
                                 preferred_element_type=jnp.float32)
        gh = jax.lax.dot_general(h.astype(jnp.bfloat16), wh,
                                 (((1,), (1,)), ((), ())),
                                 preferred_element_type=jnp.float32) + bhb
        rz = jax.nn.sigmoid(gi[:, :2 * H] + gh[:, :2 * H])
        z = rz[:, H:]
        n = jnp.tanh(gi[:, 2 * H:] + rz[:, :H] * gh[:, 2 * H:])
        return (1.0 - z) * n + z * h

    def enc_step(t):
        @pl.when(t < me)
        def _():
            for a0, leb in ((0, lebA), (SUB, lebB)):
                h = h_ref[a0:a0 + SUB, :]
                h_new = cell(ids_e, ge_ref, a0, t, whe_, bheb, h)
                h_ref[a0:a0 + SUB, :] = jnp.where(leb > t, h_new, h)

    def dec_step(t):
        @pl.when(t < md)
        def _():
            for a0, ldb in ((0, ldbA), (SUB, ldbB)):
                h = h_ref[a0:a0 + SUB, :]
                h_new = cell(ids_d, gd_ref, a0, t, whd_, bhdb, h)
                msk = ldb > t
                out[a0:a0 + SUB, t * H:(t + 1) * H] = jnp.where(
                    msk, h_new, 0.0)
                h_ref[a0:a0 + SUB, :] = jnp.where(msk, h_new, h)

        @pl.when(t >= md)
        def _():
            out[:, t * H:(t + 1) * H] = jnp.zeros((BLK, H), jnp.float32)

    h_ref[:] = jnp.zeros((BLK, H), jnp.float32)
    for t in range(L):
        enc_step(t)
    for t in range(L):
        dec_step(t)


def kernel(input, output, input_mask, output_mask, input_word_len,
           output_word_len, emb, w_ih_enc, w_hh_enc, b_ih_enc, b_hh_enc,
           w_ih_dec, w_hh_dec, b_ih_dec, b_hh_dec):
    in_len = input_word_len[:, 0]
    perm_in = jnp.argsort(-in_len)
    ids_e = jnp.take(input, perm_in, axis=0).astype(jnp.int16)
    len_e = jnp.take(in_len, perm_in).astype(jnp.int32)

    out_len = output_word_len[:, 0]
    perm_out = jnp.argsort(-out_len)
    ids_d = jnp.take(output, perm_out, axis=0).astype(jnp.int16)
    len_d = jnp.take(out_len, perm_out).astype(jnp.int32)

    maxlens = jnp.zeros((NB, 2), jnp.int32)

    full = lambda shape: pl.BlockSpec(shape, lambda i, *_: (0,) * len(shape))
    ids_spec = pl.BlockSpec((BLK, L), lambda i, *_: (i, 0))
    len_spec = pl.BlockSpec((BLK, 1), lambda i, *_: (i, 0))

    grid_spec = pltpu.PrefetchScalarGridSpec(
        num_scalar_prefetch=1,
        grid=(NB,),
        in_specs=[
            ids_spec, len_spec, ids_spec, len_spec,
            full((V, E)),
            full((3 * H, E)), full((3 * H, H)), full((1, 3 * H)),
            full((3 * H, E)), full((3 * H, H)), full((1, 3 * H)),
            full((1, 3 * H)), full((1, 3 * H)),
        ],
        out_specs=pl.BlockSpec((BLK, L * H), lambda i, *_: (i, 0)),
        scratch_shapes=[
            pltpu.VMEM((V, 3 * H), jnp.bfloat16),
            pltpu.VMEM((V, 3 * H), jnp.bfloat16),
            pltpu.VMEM((BLK, H), jnp.float32),
        ],
    )

    out_flat = pl.pallas_call(
        _body,
        grid_spec=grid_spec,
        out_shape=jax.ShapeDtypeStruct((B, L * H), jnp.float32),
    )(maxlens, ids_e, len_e[:, None], ids_d, len_d[:, None], emb,
      w_ih_enc, w_hh_enc, b_ih_enc.reshape(1, 3 * H),
      w_ih_dec, w_hh_dec, b_ih_dec.reshape(1, 3 * H),
      b_hh_enc.reshape(1, 3 * H), b_hh_dec.reshape(1, 3 * H))
    return out_flat.reshape(B, L, H)


# speedup vs baseline: 2.2957x; 1.5705x over previous
import jax
import jax.numpy as jnp
from jax.experimental import pallas as pl
from jax.experimental.pallas import tpu as pltpu

B, L, V, E, H = 16384, 20, 512, 64, 64
BLK = 2048
SUB = 1024
NB = B // BLK


def _body(lens, ids_e, len_e, ids_d, len_d, emb_t, wie, whe, bie,
          wid, whd, bid, bhe, bhd, out, ge_ref, gd_ref):
    i = pl.program_id(0)
    me = lens[i, 0]
    md = lens[i, 1]

    @pl.when(i == 0)
    def _build_tables():
        embv = emb_t[:].astype(jnp.bfloat16)
        ge_ref[:] = (jax.lax.dot_general(
            wie[:].astype(jnp.bfloat16), embv, (((1,), (0,)), ((), ())),
            preferred_element_type=jnp.float32) + bie[:]).astype(jnp.bfloat16)
        gd_ref[:] = (jax.lax.dot_general(
            wid[:].astype(jnp.bfloat16), embv, (((1,), (0,)), ((), ())),
            preferred_element_type=jnp.float32) + bid[:]).astype(jnp.bfloat16)

    iota = jax.lax.broadcasted_iota(jnp.int32, (V, 1), 0).astype(jnp.int16)
    one_bf = jnp.ones((), jnp.bfloat16)
    zero_bf = jnp.zeros((), jnp.bfloat16)

    whe_ = whe[:].astype(jnp.bfloat16)
    whd_ = whd[:].astype(jnp.bfloat16)
    bheb = jnp.broadcast_to(bhe[:], (3 * H, SUB))
    bhdb = jnp.broadcast_to(bhd[:], (3 * H, SUB))

    lensA_e = len_e[0:1, 0:SUB]
    lensB_e = len_e[0:1, SUB:BLK]
    lensA_d = len_d[0:1, 0:SUB]
    lensB_d = len_d[0:1, SUB:BLK]

    def cell(ids_ref, g_ref, a0, t, wh, bhb, h):
        idc = ids_ref[pl.ds(t, 1), 0, a0:a0 + SUB]
        oh = jnp.where(idc == iota, one_bf, zero_bf)
        gi = jax.lax.dot_general(g_ref[:], oh, (((1,), (0,)), ((), ())),
                                 preferred_element_type=jnp.float32)
        gh = jax.lax.dot_general(wh, h.astype(jnp.bfloat16),
                                 (((1,), (0,)), ((), ())),
                                 preferred_element_type=jnp.float32) + bhb
        rz = jax.nn.sigmoid(gi[:2 * H, :] + gh[:2 * H, :])
        z = rz[H:, :]
        n = jnp.tanh(gi[2 * H:, :] + rz[:H, :] * gh[2 * H:, :])
        return (1.0 - z) * n + z * h

    def enc_body(t, hs):
        hA, hB = hs
        hnA = cell(ids_e, ge_ref, 0, t, whe_, bheb, hA)
        hnB = cell(ids_e, ge_ref, SUB, t, whe_, bheb, hB)
        hA = jnp.where(lensA_e > t, hnA, hA)
        hB = jnp.where(lensB_e > t, hnB, hB)
        return hA, hB

    def dec_body(t, hs):
        hA, hB = hs
        hnA = cell(ids_d, gd_ref, 0, t, whd_, bhdb, hA)
        hnB = cell(ids_d, gd_ref, SUB, t, whd_, bhdb, hB)
        mA = lensA_d > t
        mB = lensB_d > t
        out[pl.ds(t, 1), :, 0:SUB] = jnp.where(mA, hnA, 0.0).reshape(
            1, H, SUB)
        out[pl.ds(t, 1), :, SUB:BLK] = jnp.where(mB, hnB, 0.0).reshape(
            1, H, SUB)
        hA = jnp.where(mA, hnA, hA)
        hB = jnp.where(mB, hnB, hB)
        return hA, hB

    out[:] = jnp.zeros((L, H, BLK), jnp.float32)
    h0 = jnp.zeros((H, SUB), jnp.float32)
    hs = jax.lax.fori_loop(0, me, enc_body, (h0, h0))
    jax.lax.fori_loop(0, md, dec_body, hs)


def kernel(input, output, input_mask, output_mask, input_word_len,
           output_word_len, emb, w_ih_enc, w_hh_enc, b_ih_enc, b_hh_enc,
           w_ih_dec, w_hh_dec, b_ih_dec, b_hh_dec):
    in_len = input_word_len[:, 0]
    perm_in = jnp.argsort(-in_len)
    ids_e = jnp.take(input, perm_in, axis=0).astype(jnp.int16)
    len_e = jnp.take(in_len, perm_in).astype(jnp.int32)

    out_len = output_word_len[:, 0]
    perm_out = jnp.argsort(-out_len)
    ids_d = jnp.take(output, perm_out, axis=0).astype(jnp.int16)
    len_d = jnp.take(out_len, perm_out).astype(jnp.int32)

    ids_e3 = ids_e.T.reshape(L, 1, B)
    ids_d3 = ids_d.T.reshape(L, 1, B)

    maxlens = jnp.stack([len_e[::BLK], len_d[::BLK]], axis=1).astype(jnp.int32)

    full = lambda shape: pl.BlockSpec(shape, lambda i, *_: (0,) * len(shape))
    ids_spec = pl.BlockSpec((L, 1, BLK), lambda i, *_: (0, 0, i))
    len_spec = pl.BlockSpec((1, BLK), lambda i, *_: (0, i))

    grid_spec = pltpu.PrefetchScalarGridSpec(
        num_scalar_prefetch=1,
        grid=(NB,),
        in_specs=[
            ids_spec, len_spec, ids_spec, len_spec,
            full((E, V)),
            full((3 * H, E)), full((3 * H, H)), full((3 * H, 1)),
            full((3 * H, E)), full((3 * H, H)), full((3 * H, 1)),
            full((3 * H, 1)), full((3 * H, 1)),
        ],
        out_specs=pl.BlockSpec((L, H, BLK), lambda i, *_: (0, 0, i)),
        scratch_shapes=[
            pltpu.VMEM((3 * H, V), jnp.bfloat16),
            pltpu.VMEM((3 * H, V), jnp.bfloat16),
        ],
    )

    out_t = pl.pallas_call(
        _body,
        grid_spec=grid_spec,
        out_shape=jax.ShapeDtypeStruct((L, H, B), jnp.float32),
    )(maxlens, ids_e3, len_e[None, :], ids_d3, len_d[None, :], emb.T,
      w_ih_enc, w_hh_enc, b_ih_enc.reshape(3 * H, 1),
      w_ih_dec, w_hh_dec, b_ih_dec.reshape(3 * H, 1),
      b_hh_enc.reshape(3 * H, 1), b_hh_dec.reshape(3 * H, 1))
    return jnp.transpose(out_t, (2, 0, 1))
